# baseline (speedup 1.0000x reference)
import functools

import jax
import jax.numpy as jnp
from jax import lax
from jax.experimental import pallas as pl
from jax.experimental.pallas import tpu as pltpu
from jax.experimental.pallas import tpu_sc as plsc

BATCH = 16384
EMBED_DIM = 16
LANES = 16
CHUNK = 128


def _make_sc_kernel(num_workers: int, b_per_w: int):
    n_chunks = b_per_w // CHUNK
    mesh = plsc.VectorSubcoreMesh(core_axis_name="c", subcore_axis_name="s")

    @functools.partial(
        pl.kernel,
        mesh=mesh,
        out_type=jax.ShapeDtypeStruct((BATCH,), jnp.float32),
        compiler_params=pltpu.CompilerParams(
            needs_layout_passes=False, use_tc_tiling_on_sc=False),
        scratch_types=[
            pltpu.VMEM((n_chunks, CHUNK), jnp.int32),
            pltpu.VMEM((n_chunks, CHUNK), jnp.int32),
            pltpu.VMEM((b_per_w, EMBED_DIM), jnp.float32),
            pltpu.VMEM((b_per_w, EMBED_DIM), jnp.float32),
            pltpu.VMEM((b_per_w,), jnp.float32),
            pltpu.SemaphoreType.DMA,
        ],
    )
    def sc_kernel(user_hbm, songs_hbm, utab_hbm, stab_hbm, out_hbm,
                  uidx, sidx, urows, srows, outv, sem):
        num_cores = 2
        wid = lax.axis_index("s") * num_cores + lax.axis_index("c")
        base = wid * b_per_w

        for c in range(n_chunks):
            pltpu.sync_copy(user_hbm.at[pl.ds(base + c * CHUNK, CHUNK)],
                            uidx.at[c])
            pltpu.sync_copy(songs_hbm.at[pl.ds(base + c * CHUNK, CHUNK)],
                            sidx.at[c])

        copies = []
        for c in range(n_chunks):
            copies.append(pltpu.async_copy(
                utab_hbm.at[uidx.at[c]],
                urows.at[pl.ds(c * CHUNK, CHUNK)], sem))
            copies.append(pltpu.async_copy(
                stab_hbm.at[sidx.at[c]],
                srows.at[pl.ds(c * CHUNK, CHUNK)], sem))
        for cp in copies:
            cp.wait()

        def g_body(g, carry):
            ridx = lax.iota(jnp.int32, LANES) + g * LANES
            acc = jnp.zeros((LANES,), jnp.float32)
            for d in range(EMBED_DIM):
                dv = jnp.full((LANES,), d, jnp.int32)
                uc = plsc.load_gather(urows, [ridx, dv])
                sc = plsc.load_gather(srows, [ridx, dv])
                acc = acc + uc * sc
            plsc.store_scatter(outv, [ridx], acc)
            return carry

        lax.fori_loop(0, b_per_w // LANES, g_body, 0)

        pltpu.sync_copy(outv, out_hbm.at[pl.ds(base, b_per_w)])

    return sc_kernel


def kernel(user, songs, user_table, song_table):
    info = plsc.get_sparse_core_info()
    num_workers = info.num_cores * info.num_subcores
    b_per_w = BATCH // num_workers
    sc = _make_sc_kernel(num_workers, b_per_w)
    out = sc(user.reshape(BATCH).astype(jnp.int32),
             songs.reshape(BATCH).astype(jnp.int32),
             user_table, song_table)
    return out.reshape(BATCH, 1)
